# baseline (speedup 1.0000x reference)
import jax
import jax.numpy as jnp
from jax.experimental import pallas as pl

B, A, K, MB = 256, 64, 6, 64
AF, BF, FP = 72, 10, 128
NLAYERS, NMOL = 3, 2
EPS = 1e-06
NEG = -900000000.0
MBLK = 16
N = MBLK * A
_SQ = float((1.0 + EPS) ** 0.5)


def _lbn_w(p):
    return [p['W'].T, p['b'].reshape(1, -1),
            (p['g'] / _SQ).reshape(1, -1), p['be'].reshape(1, -1)]


def _leaky(x):
    return jnp.where(x >= 0, x, 0.01 * x)


def _elu(x):
    return jnp.where(x > 0, x, jnp.exp(jnp.minimum(x, 0.0)) - 1.0)


def _tanh(x):
    xc = jnp.clip(x, -9.0, 9.0)
    x2 = xc * xc
    p = xc * (4.89352455891786e-03 + x2 * (6.37261928875436e-04 + x2 * (
        1.48572235717979e-05 + x2 * (5.12229709037114e-08 + x2 * (
            -8.60467152213735e-11 + x2 * (2.00018790482477e-13 + x2 * (
                -2.76076847742355e-16)))))))
    q = 4.89352518554385e-03 + x2 * (2.26843463243900e-03 + x2 * (
        1.18534705686654e-04 + x2 * 1.19825839466702e-06))
    return jnp.where(jnp.abs(x) < 0.0004, x, p / q)


def _body(af_ref, bf_ref, ia_ref, ib_ref, ik_ref, mk_ref, *refs):
    o_ref = refs[-1]
    w = [r[...] for r in refs[:-1]]
    it = iter(w)

    def nxt(n_):
        return [next(it) for _ in range(n_)]

    f32 = jnp.float32
    bf16 = jnp.bfloat16
    dot = lambda a, b_: jnp.dot(a, b_, preferred_element_type=f32)

    def lbn(x, ws):
        Wt, b_, s_, be_ = ws
        return (dot(x, Wt) + b_) * s_ + be_

    x = af_ref[...].reshape(N, AF)
    x = jnp.maximum(lbn(x, nxt(4)), 0.0)
    atom_fp = jnp.maximum(lbn(x, nxt(4)), 0.0)
    y = bf_ref[...].reshape(N, BF)
    y = jnp.maximum(lbn(y, nxt(4)), 0.0)
    bond_fp = jnp.maximum(lbn(y, nxt(4)), 0.0)

    def gather_km(ohs, table):
        hi = table.astype(bf16)
        mid = (table - hi.astype(f32)).astype(bf16)
        lo = (table - hi.astype(f32) - mid.astype(f32)).astype(bf16)
        outs = [[] for _ in range(K)]
        for m in range(MBLK):
            sl = slice(m * A, (m + 1) * A)
            t3 = jnp.concatenate([hi[sl], mid[sl], lo[sl]], axis=0)
            g = jnp.dot(ohs[m], t3, preferred_element_type=f32)
            for k in range(K):
                outs[k].append(g[k * A:(k + 1) * A])
        return [jnp.concatenate(c, axis=0) for c in outs]

    oh_a = [ia_ref[m] for m in range(MBLK)]
    oh_b = [ib_ref[m] for m in range(MBLK)]
    anei = gather_km(oh_a, atom_fp)
    bnei = gather_km(oh_b, bond_fp)

    nei_w1, nei_w2 = nxt(4), nxt(4)
    nei = []
    for k in range(K):
        a_k, b_k = anei[k], bnei[k]
        mix = a_k + b_k - a_k * b_k
        nb = jnp.concatenate([a_k, b_k, mix], axis=1)
        h1 = jnp.maximum(lbn(nb, nei_w1), 0.0)
        nei.append(jnp.maximum(lbn(h1, nei_w2), 0.0))

    row_k = jax.lax.broadcasted_iota(jnp.int32, (3 * K, K * FP), 0) % K
    col_k = jax.lax.broadcasted_iota(jnp.int32, (3 * K, K * FP), 1) // FP
    rep_mat = (row_k == col_k).astype(bf16)

    idxs = ik_ref[...].reshape(N, K)
    att_mask = (idxs != A - 1).astype(f32)
    smask = jnp.where(idxs == A - 1, NEG, 0.0).astype(f32)

    for _ in range(NLAYERS):
        (wal, Wbd, bal, sal, beal) = nxt(5)
        att_w = nxt(4)
        (Wih, bih, Whh, bhh) = nxt(4)
        nei_stack = jnp.concatenate(nei, axis=1)
        s = dot(atom_fp, wal) + dot(nei_stack, Wbd)
        s = (s + bal) * sal + beal
        s = _leaky(s) + smask
        s = s - jnp.max(s, axis=1, keepdims=True)
        e = jnp.exp(s)
        wgt = e / jnp.sum(e, axis=1, keepdims=True) * att_mask
        whi = wgt.astype(bf16)
        wmid = (wgt - whi.astype(f32)).astype(bf16)
        wlo = (wgt - whi.astype(f32) - wmid.astype(f32)).astype(bf16)
        wrep = jnp.dot(jnp.concatenate([whi, wmid, wlo], axis=1), rep_mat,
                       preferred_element_type=f32)
        Wt_a, b_a, s_a, be_a = att_w
        acc = None
        for k in range(K):
            att_k = (dot(nei[k], Wt_a) + b_a) * s_a + be_a
            term = wrep[:, k * FP:(k + 1) * FP] * att_k
            acc = term if acc is None else acc + term
        ctx = _elu(acc)
        gi = dot(ctx, Wih) + bih
        gh = dot(atom_fp, Whh) + bhh
        r = jax.nn.sigmoid(gi[:, :FP] + gh[:, :FP])
        z = jax.nn.sigmoid(gi[:, FP:2 * FP] + gh[:, FP:2 * FP])
        n = jnp.tanh(gi[:, 2 * FP:] + r * gh[:, 2 * FP:])
        new_atom = (1.0 - z) * n + z * atom_fp
        act = jnp.maximum(new_atom, 0.0)
        nei = gather_km(oh_a, act)
        atom_fp = new_atom

    mask3 = mk_ref[...]
    molmask = jnp.where(mask3 == 0.0, NEG, 0.0)
    afp3 = atom_fp.reshape(MBLK, A, FP)
    sup = jnp.sum(afp3 * mask3, axis=1)
    (wsm, wam, bam, sam, beam) = nxt(5)
    matt_w = nxt(4)
    (Wihm, bihm, Whhm, bhhm) = nxt(4)
    for _ in range(NMOL):
        s_at = dot(atom_fp, wam).reshape(MBLK, A, 1)
        s_sup = dot(sup, wsm).reshape(MBLK, 1, 1)
        s = (s_at + s_sup + bam) * sam + beam
        s = _leaky(s) + molmask
        s = s - jnp.max(s, axis=1, keepdims=True)
        e = jnp.exp(s)
        wgt = e / jnp.sum(e, axis=1, keepdims=True) * mask3
        att = lbn(atom_fp, matt_w)
        ctx = _elu(jnp.sum(att.reshape(MBLK, A, FP) * wgt, axis=1))
        gi = dot(ctx, Wihm) + bihm
        gh = dot(sup, Whhm) + bhhm
        r = jax.nn.sigmoid(gi[:, :FP] + gh[:, :FP])
        z = jax.nn.sigmoid(gi[:, FP:2 * FP] + gh[:, FP:2 * FP])
        n = jnp.tanh(gi[:, 2 * FP:] + r * gh[:, 2 * FP:])
        sup = (1.0 - z) * n + z * sup

    (W1d, b1d, W2d, b2d, W3d, b3d) = nxt(6)
    mol_fp = jnp.maximum(sup, 0.0)
    h = jnp.maximum(dot(mol_fp, W1d) + b1d, 0.0)
    h = jnp.maximum(dot(h, W2d) + b2d, 0.0)
    o_ref[...] = dot(h, W3d) + b3d


def kernel(atom_features, bond_features, atom_neighbor_list,
           bond_neighbor_list, atom_mask, params):
    f32 = jnp.float32
    wlist = []
    for p in params['atom_fc'] + params['bond_fc']:
        wlist += _lbn_w(p)
    for p in params['nei_fc']:
        wlist += _lbn_w(p)
    eye_k = jnp.eye(K, dtype=f32)
    for lp in params['layers']:
        al = lp['align']
        wt = al['W'].T
        wlist += [wt[:FP], jnp.kron(eye_k, wt[FP:]),
                  al['b'].reshape(1, 1), (al['g'] / _SQ).reshape(1, 1),
                  al['be'].reshape(1, 1)]
        wlist += _lbn_w(lp['attend'])
        g = lp['gru']
        wlist += [g['Wih'].T, g['bih'].reshape(1, -1),
                  g['Whh'].T, g['bhh'].reshape(1, -1)]
    mp = params['mol']
    al = mp['align']
    wt = al['W'].T
    wlist += [wt[:FP], wt[FP:], al['b'].reshape(1, 1),
              (al['g'] / _SQ).reshape(1, 1), al['be'].reshape(1, 1)]
    wlist += _lbn_w(mp['attend'])
    g = mp['gru']
    wlist += [g['Wih'].T, g['bih'].reshape(1, -1),
              g['Whh'].T, g['bhh'].reshape(1, -1)]
    d = params['dnn']
    wlist += [d['W1'].T, d['b1'].reshape(1, -1),
              d['W2'].T, d['b2'].reshape(1, -1),
              d['W3'].T, d['b3'].reshape(1, -1)]
    wlist = [w.astype(f32) for w in wlist]

    lane_val = (jnp.arange(3 * A, dtype=jnp.int32) & (A - 1))
    ia = jnp.transpose(atom_neighbor_list, (0, 2, 1)).reshape(B, K * A)
    ib = jnp.transpose(bond_neighbor_list, (0, 2, 1)).reshape(B, K * A)
    oh_a = (ia[:, :, None] == lane_val).astype(jnp.bfloat16)
    oh_b = (ib[:, :, None] == lane_val).astype(jnp.bfloat16)
    mask3 = atom_mask[..., None]

    in_specs = [
        pl.BlockSpec((MBLK, A, AF), lambda i: (i, 0, 0)),
        pl.BlockSpec((MBLK, MB, BF), lambda i: (i, 0, 0)),
        pl.BlockSpec((MBLK, K * A, 3 * A), lambda i: (i, 0, 0)),
        pl.BlockSpec((MBLK, K * A, 3 * A), lambda i: (i, 0, 0)),
        pl.BlockSpec((MBLK, A, K), lambda i: (i, 0, 0)),
        pl.BlockSpec((MBLK, A, 1), lambda i: (i, 0, 0)),
    ] + [pl.BlockSpec(wa.shape, lambda i, nd=wa.ndim: (0,) * nd)
         for wa in wlist]

    return pl.pallas_call(
        _body,
        grid=(B // MBLK,),
        in_specs=in_specs,
        out_specs=pl.BlockSpec((MBLK, 1), lambda i: (i, 0)),
        out_shape=jax.ShapeDtypeStruct((B, 1), f32),
    )(atom_features, bond_features, oh_a, oh_b, atom_neighbor_list, mask3,
      *wlist)
